# baseline (speedup 1.0000x reference)
import functools

import jax
import jax.numpy as jnp
from jax import lax
from jax.experimental import pallas as pl
from jax.experimental.pallas import tpu as pltpu
from jax.experimental.pallas import tpu_sc as plsc

_BATCH = 16384
_D = 32
_H1 = 64
_H2 = 32
_NCLS = 2
_NC = 2
_NS = 16
_NW = _NC * _NS
_ROWS = 1000000
_CH = 512
_NCID = (_ROWS + _CH - 1) // _CH
_KMAX = (_NCID + _NW - 1) // _NW
_PHYS_LANES = 1000064
_LAST_START = _PHYS_LANES - _CH
_SROWS = 64
_LISTN = _BATCH + 16

_sc_mesh = plsc.VectorSubcoreMesh(core_axis_name="c", subcore_axis_name="s")


@functools.partial(
    pl.kernel,
    mesh=_sc_mesh,
    out_type=(
        jax.ShapeDtypeStruct((_BATCH + 1, 128), jnp.float32),
        jax.ShapeDtypeStruct((_BATCH + 1, 128), jnp.float32),
    ),
    scratch_types=[
        pltpu.VMEM((1024,), jnp.int32),
        pltpu.VMEM((_LISTN,), jnp.int32),
        pltpu.VMEM((_LISTN,), jnp.int32),
        pltpu.VMEM((_LISTN,), jnp.int32),
        pltpu.VMEM((_LISTN,), jnp.int32),
        pltpu.VMEM((2, _D, _CH), jnp.float32),
        pltpu.VMEM((_SROWS, 128), jnp.float32),
        pltpu.VMEM((_SROWS,), jnp.int32),
        pltpu.SemaphoreType.DMA,
        pltpu.SemaphoreType.DMA,
        pltpu.SemaphoreType.DMA,
    ],
    compiler_params=pltpu.CompilerParams(use_tc_tiling_on_sc=True,
                                         needs_layout_passes=False),
)
def _sc_gather(ut_t, it_t, uidx, iidx, ue_out, ie_out,
               ipiece, blist, rlist, cbp, crv, bufs, staging, sidx,
               ssem, fsem, csem):
    w = lax.axis_index("s") * _NC + lax.axis_index("c")
    iota = jnp.arange(16, dtype=jnp.int32)
    rows_lo = iota
    rows_hi = iota + 16
    lane0 = iota == 0
    trash = jnp.full((16,), _BATCH, dtype=jnp.int32)

    def _pc(m):
        c = plsc.all_reduce_population_count(m)
        return c[0] if getattr(c, "ndim", 0) else c

    def _reset_sidx():
        for t in range(_SROWS // 16):
            sidx[pl.ds(t * 16, 16)] = trash

    _reset_sidx()

    def _run_table(tab, idx_hbm, out_hbm):
        def _piece(p, cnt):
            cp = pltpu.async_copy(idx_hbm.at[pl.ds(p * 1024, 1024)], ipiece, ssem)
            cp.wait()

            def _seg(s, cnt):
                rv = ipiece[pl.ds(s * 16, 16)]
                m = ((rv >> 9) & (_NW - 1)) == w
                bp = (p * 1024 + s * 16) + iota
                plsc.store_compressed(rlist.at[pl.ds(cnt, 16)], rv, mask=m)
                plsc.store_compressed(blist.at[pl.ds(cnt, 16)], bp, mask=m)
                return cnt + _pc(m)

            return lax.fori_loop(0, 64, _seg, cnt)

        cnt = lax.fori_loop(0, 16, _piece, jnp.int32(0))
        nseg = (cnt + 15) >> 4

        def _cid(k):
            return k * _NW + w

        def _start_of(cid):
            return pl.multiple_of(jnp.minimum(cid * _CH, _LAST_START), 128)

        def _fire(k, buf):
            @pl.when(_cid(k) < _NCID)
            def _():
                pltpu.make_async_copy(
                    tab.at[:, pl.ds(_start_of(_cid(k)), _CH)], buf, fsem
                ).start()

        def _wait_fetch(k, buf):
            @pl.when(_cid(k) < _NCID)
            def _():
                pltpu.make_async_copy(
                    tab.at[:, pl.ds(0, _CH)], buf, fsem
                ).wait()

        def _flush():
            pltpu.async_copy(staging, out_hbm.at[sidx], csem).wait()
            _reset_sidx()
            return jnp.int32(0)

        def _process(k, buf, fill0):
            start = _start_of(_cid(k))

            def _seg2(s, cc):
                rv = rlist[pl.ds(s * 16, 16)]
                bv = blist[pl.ds(s * 16, 16)]
                pos = s * 16 + iota
                m = (pos < cnt) & ((rv >> 14) == k)
                plsc.store_compressed(crv.at[pl.ds(cc, 16)], rv, mask=m)
                plsc.store_compressed(cbp.at[pl.ds(cc, 16)], bv, mask=m)
                return cc + _pc(m)

            cc = lax.fori_loop(0, nseg, _seg2, jnp.int32(0))

            def _item(j, fill):
                r = crv[pl.ds(j, 16)][0]
                bp = cbp[pl.ds(j, 16)][0]
                lv = jnp.full((16,), r - start, dtype=jnp.int32)
                v0 = plsc.load_gather(buf, [rows_lo, lv])
                v1 = plsc.load_gather(buf, [rows_hi, lv])
                fv = jnp.full((16,), fill, dtype=jnp.int32)
                plsc.store_scatter(staging, [fv, iota], v0)
                plsc.store_scatter(staging, [fv, iota + 16], v1)
                plsc.store_scatter(sidx, [fv],
                                   jnp.full((16,), bp, dtype=jnp.int32),
                                   mask=lane0)
                fill = fill + 1
                return lax.cond(fill == _SROWS, _flush, lambda: fill)

            return lax.fori_loop(0, cc, _item, fill0)

        def _process_guarded(k, buf, fill):
            return lax.cond(_cid(k) < _NCID,
                            lambda: _process(k, buf, fill),
                            lambda: fill)

        _fire(0, bufs.at[0])

        def _pair(h, fill):
            k = 2 * h
            _fire(k + 1, bufs.at[1])
            _wait_fetch(k, bufs.at[0])
            fill = _process_guarded(k, bufs.at[0], fill)
            _fire(k + 2, bufs.at[0])
            _wait_fetch(k + 1, bufs.at[1])
            fill = _process_guarded(k + 1, bufs.at[1], fill)
            return fill

        fill = lax.fori_loop(0, _KMAX // 2, _pair, jnp.int32(0))
        lax.cond(fill > 0, _flush, lambda: fill)

    _run_table(ut_t, uidx, ue_out)
    _run_table(it_t, iidx, ie_out)


_BM = 2048
_GRID = _BATCH // _BM


def _mlp_body(ue, ie, w1u, w1i, b1, w2, b2, w3, b3, out):
    x = jnp.dot(ue[:, :_D], w1u[...], preferred_element_type=jnp.float32)
    x = x + jnp.dot(ie[:, :_D], w1i[...], preferred_element_type=jnp.float32)
    x = jnp.maximum(x + b1[...], 0.0)
    x = jnp.maximum(jnp.dot(x, w2[...], preferred_element_type=jnp.float32) + b2[...], 0.0)
    x = jnp.maximum(jnp.dot(x, w3[...], preferred_element_type=jnp.float32) + b3[...], 0.0)
    out[...] = x


_mlp = pl.pallas_call(
    _mlp_body,
    grid=(_GRID,),
    in_specs=[
        pl.BlockSpec((_BM, 128), lambda i: (i, 0)),
        pl.BlockSpec((_BM, 128), lambda i: (i, 0)),
        pl.BlockSpec((_D, _H1), lambda i: (0, 0)),
        pl.BlockSpec((_D, _H1), lambda i: (0, 0)),
        pl.BlockSpec((1, _H1), lambda i: (0, 0)),
        pl.BlockSpec((_H1, _H2), lambda i: (0, 0)),
        pl.BlockSpec((1, _H2), lambda i: (0, 0)),
        pl.BlockSpec((_H2, _NCLS), lambda i: (0, 0)),
        pl.BlockSpec((1, _NCLS), lambda i: (0, 0)),
    ],
    out_specs=pl.BlockSpec((_BM, _NCLS), lambda i: (i, 0)),
    out_shape=jax.ShapeDtypeStruct((_BATCH, _NCLS), jnp.float32),
)


def kernel(user_input, item_input, user_table, item_table, W1, b1, W2, b2, W3, b3):
    ue, ie = _sc_gather(user_table.T, item_table.T,
                        user_input.astype(jnp.int32),
                        item_input.astype(jnp.int32))
    return _mlp(ue, ie, W1[:_D], W1[_D:], b1.reshape(1, _H1),
                W2, b2.reshape(1, _H2), W3, b3.reshape(1, _NCLS))
